# baseline (speedup 1.0000x reference)
import jax
import jax.numpy as jnp
from jax.experimental import pallas as pl
from jax.experimental.pallas import tpu as pltpu

_C = 1.0
_EPS = 1e-5
_MIN_NORM = 1e-15


def _artanh(x):
    x = jnp.clip(x, -1 + 1e-7, 1 - 1e-7)
    return 0.5 * jnp.log((1 + x) / (1 - x))


def _row_norm(x):
    return jnp.clip(jnp.sqrt(jnp.sum(x * x, axis=-1, keepdims=True)), _MIN_NORM, None)


def _proj(x):
    norm = _row_norm(x)
    maxnorm = (1 - _EPS)
    return jnp.where(norm > maxnorm, x / norm * maxnorm, x)


def _expmap0(u):
    u_norm = _row_norm(u)
    return jnp.tanh(u_norm) * u / u_norm


def _hgc_kernel(adj_ref, x_ref, w_ref, b_ref, bo_ref, out_ref, xt_ref):
    i = pl.program_id(0)

    @pl.when(i == 0)
    def _stage1():
        x = x_ref[...]
        w = w_ref[...]
        x_norm = _row_norm(x)
        mx = jax.lax.dot_general(
            x, w, (((1,), (1,)), ((), ())), preferred_element_type=jnp.float32
        )
        mx_norm = _row_norm(mx)
        res_c = jnp.tanh(mx_norm / x_norm * _artanh(x_norm)) * mx / mx_norm
        zero_row = jnp.all(mx == 0, axis=-1, keepdims=True)
        res = _proj(jnp.where(zero_row, jnp.zeros_like(res_c), res_c))
        hb = _proj(_expmap0(b_ref[...]))
        x2 = jnp.sum(res * res, axis=-1, keepdims=True)
        y2 = jnp.sum(hb * hb, axis=-1, keepdims=True)
        xy = jnp.sum(res * hb, axis=-1, keepdims=True)
        num = (1 + 2 * xy + y2) * res + (1 - x2) * hb
        den = jnp.clip(1 + 2 * xy + x2 * y2, _MIN_NORM, None)
        res2 = _proj(num / den)
        p_norm = _row_norm(res2)
        xt_ref[...] = (res2 / p_norm * _artanh(p_norm)).astype(jnp.bfloat16)

    @pl.when(i > 0)
    def _stage2():
        a = adj_ref[...].astype(jnp.bfloat16)
        s = jnp.dot(a, xt_ref[...], preferred_element_type=jnp.float32)
        out_ref[...] = _proj(_expmap0(s)) + bo_ref[...]


def kernel(adjacency, input_feature, W, b_lin, bias_out):
    N, din = input_feature.shape
    dout = W.shape[0]
    BM = 512
    nblk = N // BM
    b2 = b_lin.reshape(1, dout).astype(jnp.float32)
    bo2 = bias_out.reshape(1, dout).astype(jnp.float32)
    return pl.pallas_call(
        _hgc_kernel,
        grid=(nblk + 1,),
        in_specs=[
            pl.BlockSpec((BM, N), lambda i: (jnp.maximum(i - 1, 0), 0)),
            pl.BlockSpec((N, din), lambda i: (0, 0)),
            pl.BlockSpec((dout, din), lambda i: (0, 0)),
            pl.BlockSpec((1, dout), lambda i: (0, 0)),
            pl.BlockSpec((1, dout), lambda i: (0, 0)),
        ],
        out_specs=pl.BlockSpec((BM, dout), lambda i: (jnp.maximum(i - 1, 0), 0)),
        out_shape=jax.ShapeDtypeStruct((N, dout), jnp.float32),
        scratch_shapes=[pltpu.VMEM((N, dout), jnp.bfloat16)],
    )(adjacency, input_feature, W, b2, bo2)


# speedup vs baseline: 1.1466x; 1.1466x over previous
import jax
import jax.numpy as jnp
from jax.experimental import pallas as pl
from jax.experimental.pallas import tpu as pltpu

_EPS = 1e-5
_MIN_NORM = 1e-15
_MAXNORM = 1.0 - _EPS


def _artanh(x):
    x = jnp.clip(x, -1 + 1e-7, 1 - 1e-7)
    return 0.5 * jnp.log((1 + x) / (1 - x))


def _rnorm2(x):
    return jnp.sum(x * x, axis=-1, keepdims=True)


def _clipn(n):
    return jnp.clip(n, _MIN_NORM, None)


def _hgc_kernel(adj_ref, x_ref, w_ref, b_ref, bo_ref, out_ref, xt_ref):
    i = pl.program_id(0)

    @pl.when(i == 0)
    def _stage1():
        x = x_ref[...]
        w = w_ref[...]
        n_x = _clipn(jnp.sqrt(_rnorm2(x)))
        mx = jax.lax.dot_general(
            x, w, (((1,), (1,)), ((), ())), preferred_element_type=jnp.float32
        )
        n_mx = _clipn(jnp.sqrt(_rnorm2(mx)))
        t1 = jnp.tanh(n_mx / n_x * _artanh(n_x))
        s1 = t1 / n_mx
        s2 = jnp.where(t1 > _MAXNORM, _MAXNORM / t1, 1.0)
        sr = s1 * s2
        r = jnp.minimum(t1, _MAXNORM)
        b = b_ref[...]
        n_b = _clipn(jnp.sqrt(_rnorm2(b)))
        eb = jnp.tanh(n_b) * b / n_b
        n_eb = _clipn(jnp.sqrt(_rnorm2(eb)))
        hb = jnp.where(n_eb > _MAXNORM, eb / n_eb * _MAXNORM, eb)
        y2 = _rnorm2(hb)
        xy = jnp.sum(mx * hb, axis=-1, keepdims=True) * sr
        x2 = r * r
        alpha = 1 + 2 * xy + y2
        beta = 1 - x2
        den = _clipn(1 + 2 * xy + x2 * y2)
        a0 = alpha / den
        b0 = beta / den
        ma_n2 = a0 * a0 * x2 + 2 * a0 * b0 * xy + b0 * b0 * y2
        n_ma = _clipn(jnp.sqrt(ma_n2))
        s3 = jnp.where(n_ma > _MAXNORM, _MAXNORM / n_ma, 1.0)
        n2 = _clipn(jnp.minimum(n_ma, _MAXNORM))
        sc = s3 * _artanh(n2) / n2
        aa = a0 * sr * sc
        bb = b0 * sc
        xt_ref[...] = (aa * mx + bb * hb).astype(jnp.bfloat16)

    @pl.when(i > 0)
    def _stage2():
        a = adj_ref[...].astype(jnp.bfloat16)
        s = jnp.dot(a, xt_ref[...], preferred_element_type=jnp.float32)
        n = _clipn(jnp.sqrt(_rnorm2(s)))
        t = jnp.tanh(n)
        f = jnp.where(t > _MAXNORM, _MAXNORM / n, t / n)
        out_ref[...] = s * f + bo_ref[...]


def kernel(adjacency, input_feature, W, b_lin, bias_out):
    N, din = input_feature.shape
    dout = W.shape[0]
    BM = 512
    nblk = N // BM
    b2 = b_lin.reshape(1, dout).astype(jnp.float32)
    bo2 = bias_out.reshape(1, dout).astype(jnp.float32)
    return pl.pallas_call(
        _hgc_kernel,
        grid=(nblk + 1,),
        in_specs=[
            pl.BlockSpec((BM, N), lambda i: (jnp.maximum(i - 1, 0), 0)),
            pl.BlockSpec((N, din), lambda i: (0, 0)),
            pl.BlockSpec((dout, din), lambda i: (0, 0)),
            pl.BlockSpec((1, dout), lambda i: (0, 0)),
            pl.BlockSpec((1, dout), lambda i: (0, 0)),
        ],
        out_specs=pl.BlockSpec((BM, dout), lambda i: (jnp.maximum(i - 1, 0), 0)),
        out_shape=jax.ShapeDtypeStruct((N, dout), jnp.float32),
        scratch_shapes=[pltpu.VMEM((N, dout), jnp.bfloat16)],
    )(adjacency, input_feature, W, b2, bo2)
